# baseline (speedup 1.0000x reference)
import functools

import jax
import jax.numpy as jnp
from jax import lax
from jax.experimental import pallas as pl
from jax.experimental.pallas import tpu as pltpu
from jax.experimental.pallas import tpu_sc as plsc

_NC, _NS, _L = 2, 16, 16
_NW = _NC * _NS
_C = 16


def _log2(n):
    b = n.bit_length() - 1
    assert (1 << b) == n
    return b


def _make_sc_kernel(B, dec, enc):
    rows = B * dec
    rpw = rows // _NW
    n_chunks = rpw // _C
    dec_shift, dec_mask = _log2(dec), dec - 1
    mesh = plsc.VectorSubcoreMesh(core_axis_name="c", subcore_axis_name="s")

    @functools.partial(
        pl.kernel,
        out_type=jax.ShapeDtypeStruct((B, dec, enc), jnp.float32),
        mesh=mesh,
        compiler_params=pltpu.CompilerParams(needs_layout_passes=False),
        scratch_types=[
            pltpu.VMEM((_C, enc), jnp.float32),
            pltpu.VMEM((_C, enc), jnp.float32),
            pltpu.VMEM((_C, enc), jnp.float32),
            pltpu.VMEM((rpw,), jnp.int32),
            pltpu.SemaphoreType.DMA,
            pltpu.SemaphoreType.DMA,
            pltpu.SemaphoreType.DMA,
        ],
    )
    def sc_kernel(idx_hbm, out_hbm, tmpl_a, tmpl_b, tmpl_c, idx_v, sem_a, sem_b, sem_c):
        wid = lax.axis_index("c") * _NS + lax.axis_index("s")
        row_base = wid * rpw
        b = lax.shift_right_logical(row_base, dec_shift)
        d0 = pl.multiple_of(lax.bitwise_and(row_base, dec_mask), rpw)

        ninf = jnp.full((_L,), -jnp.inf, jnp.float32)
        one = jnp.full((_L,), 1.0, jnp.float32)
        rowiota = lax.iota(jnp.int32, _L)

        def fill(r, carry):
            for t in (tmpl_a, tmpl_b, tmpl_c):
                for u in range(enc // _L):
                    t[r, pl.ds(u * _L, _L)] = ninf
            return carry

        lax.fori_loop(0, _C, fill, 0)

        pltpu.sync_copy(idx_hbm.at[pl.ds(pl.multiple_of(row_base, rpw), rpw)], idx_v)

        nbuf = 3
        bufs = (tmpl_a, tmpl_b, tmpl_c)
        sems = (sem_a, sem_b, sem_c)
        copies = [None] * n_chunks
        for k in range(n_chunks):
            t, sem = bufs[k % nbuf], sems[k % nbuf]
            if k >= nbuf:
                copies[k - nbuf].wait()
                oldcol = idx_v[pl.ds((k - nbuf) * _C, _L)]
                plsc.store_scatter(t, [rowiota, oldcol], ninf)
            col = idx_v[pl.ds(k * _C, _L)]
            plsc.store_scatter(t, [rowiota, col], one)
            dst = out_hbm.at[b, pl.ds(d0 + k * _C, _C)]
            copies[k] = pltpu.async_copy(t, dst, sem)
        for j in range(nbuf):
            copies[n_chunks - nbuf + j].wait()

    return sc_kernel


def kernel(decoder_states, encoder_states, step, input_lengths, provided_attention):
    B, dec_seqlen = provided_attention.shape
    enc_seqlen = encoder_states.shape[1]
    idx = jnp.asarray(provided_attention, jnp.int32).reshape(B * dec_seqlen)
    return _make_sc_kernel(B, dec_seqlen, enc_seqlen)(idx)


# speedup vs baseline: 1.0807x; 1.0807x over previous
import functools

import jax
import jax.numpy as jnp
from jax import lax
from jax.experimental import pallas as pl
from jax.experimental.pallas import tpu as pltpu
from jax.experimental.pallas import tpu_sc as plsc

_NC, _NS, _L = 2, 16, 16
_NW = _NC * _NS
_C = 16


def _log2(n):
    b = n.bit_length() - 1
    assert (1 << b) == n
    return b


def _make_sc_kernel(B, dec, enc):
    rows = B * dec
    rpw = rows // _NW
    n_chunks = rpw // _C
    dec_shift, dec_mask = _log2(dec), dec - 1
    mesh = plsc.VectorSubcoreMesh(core_axis_name="c", subcore_axis_name="s")

    @functools.partial(
        pl.kernel,
        out_type=jax.ShapeDtypeStruct((B, dec, enc), jnp.float32),
        mesh=mesh,
        compiler_params=pltpu.CompilerParams(
            needs_layout_passes=False,
            disable_bounds_checks=True,
        ),
        scratch_types=[
            pltpu.VMEM((_C, enc), jnp.float32),
            pltpu.VMEM((_C, enc), jnp.float32),
            pltpu.VMEM((rpw,), jnp.int32),
            pltpu.SemaphoreType.DMA,
            pltpu.SemaphoreType.DMA,
        ],
    )
    def sc_kernel(idx_hbm, out_hbm, tmpl_a, tmpl_b, idx_v, sem_a, sem_b):
        wid = lax.axis_index("c") * _NS + lax.axis_index("s")
        row_base = wid * rpw
        b = lax.shift_right_logical(row_base, dec_shift)
        d0 = pl.multiple_of(lax.bitwise_and(row_base, dec_mask), rpw)

        ninf = jnp.full((_L,), -jnp.inf, jnp.float32)
        one = jnp.full((_L,), 1.0, jnp.float32)
        rowiota = lax.iota(jnp.int32, _L)

        idx_copy = pltpu.async_copy(
            idx_hbm.at[pl.ds(pl.multiple_of(row_base, rpw), rpw)], idx_v, sem_b
        )

        def fill(t):
            def body(r, carry):
                for u in range(enc // _L):
                    t[r, pl.ds(u * _L, _L)] = ninf
                return carry

            lax.fori_loop(0, _C, body, 0)

        def dst_of(k):
            return out_hbm.at[b, pl.ds(d0 + k * _C, _C)]

        fill(tmpl_a)
        idx_copy.wait()
        plsc.store_scatter(tmpl_a, [rowiota, idx_v[pl.ds(0, _L)]], one)
        copies = [None] * n_chunks
        copies[0] = pltpu.async_copy(tmpl_a, dst_of(0), sem_a)
        fill(tmpl_b)

        bufs = (tmpl_a, tmpl_b)
        sems = (sem_a, sem_b)
        for k in range(1, n_chunks):
            t, sem = bufs[k % 2], sems[k % 2]
            if k >= 2:
                copies[k - 2].wait()
                oldcol = idx_v[pl.ds((k - 2) * _C, _L)]
                plsc.store_scatter(t, [rowiota, oldcol], ninf)
            col = idx_v[pl.ds(k * _C, _L)]
            plsc.store_scatter(t, [rowiota, col], one)
            copies[k] = pltpu.async_copy(t, dst_of(k), sem)
        copies[n_chunks - 2].wait()
        copies[n_chunks - 1].wait()

    return sc_kernel


def kernel(decoder_states, encoder_states, step, input_lengths, provided_attention):
    B, dec_seqlen = provided_attention.shape
    enc_seqlen = encoder_states.shape[1]
    idx = jnp.asarray(provided_attention, jnp.int32).reshape(B * dec_seqlen)
    return _make_sc_kernel(B, dec_seqlen, enc_seqlen)(idx)


# speedup vs baseline: 1.0848x; 1.0037x over previous
import functools

import jax
import jax.numpy as jnp
from jax import lax
from jax.experimental import pallas as pl
from jax.experimental.pallas import tpu as pltpu
from jax.experimental.pallas import tpu_sc as plsc

_NC, _NS, _L = 2, 16, 16
_NW = _NC * _NS
_C = 16


def _log2(n):
    b = n.bit_length() - 1
    assert (1 << b) == n
    return b


def _make_sc_kernel(B, dec, enc):
    rows = B * dec
    rpw = rows // _NW
    n_chunks = rpw // _C
    dec_shift, dec_mask = _log2(dec), dec - 1
    mesh = plsc.VectorSubcoreMesh(core_axis_name="c", subcore_axis_name="s")

    @functools.partial(
        pl.kernel,
        out_type=jax.ShapeDtypeStruct((B, dec, enc), jnp.float32),
        mesh=mesh,
        compiler_params=pltpu.CompilerParams(
            needs_layout_passes=False,
            disable_bounds_checks=True,
            skip_device_barrier=True,
        ),
        scratch_types=[
            pltpu.VMEM((_C, enc), jnp.float32),
            pltpu.VMEM((_C, enc), jnp.float32),
            pltpu.VMEM((B, rpw), jnp.int32),
            pltpu.SemaphoreType.DMA,
            pltpu.SemaphoreType.DMA,
        ],
    )
    def sc_kernel(idx_hbm, out_hbm, tmpl_a, tmpl_b, idx_v, sem_a, sem_b):
        wid = lax.axis_index("c") * _NS + lax.axis_index("s")
        row_base = wid * rpw
        b = lax.shift_right_logical(row_base, dec_shift)
        d0 = pl.multiple_of(lax.bitwise_and(row_base, dec_mask), rpw)

        ninf = jnp.full((_L,), -jnp.inf, jnp.float32)
        one = jnp.full((_L,), 1.0, jnp.float32)
        rowiota = lax.iota(jnp.int32, _L)

        idx_copy = pltpu.async_copy(
            idx_hbm.at[pl.ds(0, B), pl.ds(d0, rpw)], idx_v, sem_b
        )

        def fill(t):
            def body(r, carry):
                for u in range(enc // _L):
                    t[r, pl.ds(u * _L, _L)] = ninf
                return carry

            lax.fori_loop(0, _C, body, 0)

        def dst_of(k):
            return out_hbm.at[b, pl.ds(d0 + k * _C, _C)]

        fill(tmpl_a)
        idx_copy.wait()
        plsc.store_scatter(tmpl_a, [rowiota, idx_v[b, pl.ds(0, _L)]], one)
        copies = [None] * n_chunks
        copies[0] = pltpu.async_copy(tmpl_a, dst_of(0), sem_a)
        fill(tmpl_b)

        bufs = (tmpl_a, tmpl_b)
        sems = (sem_a, sem_b)
        for k in range(1, n_chunks):
            t, sem = bufs[k % 2], sems[k % 2]
            if k >= 2:
                copies[k - 2].wait()
                oldcol = idx_v[b, pl.ds((k - 2) * _C, _L)]
                plsc.store_scatter(t, [rowiota, oldcol], ninf)
            col = idx_v[b, pl.ds(k * _C, _L)]
            plsc.store_scatter(t, [rowiota, col], one)
            copies[k] = pltpu.async_copy(t, dst_of(k), sem)
        copies[n_chunks - 2].wait()
        copies[n_chunks - 1].wait()

    return sc_kernel


def kernel(decoder_states, encoder_states, step, input_lengths, provided_attention):
    B, dec_seqlen = provided_attention.shape
    enc_seqlen = encoder_states.shape[1]
    idx = jnp.asarray(provided_attention, jnp.int32)
    return _make_sc_kernel(B, dec_seqlen, enc_seqlen)(idx)
